# baseline (speedup 1.0000x reference)
import jax
import jax.numpy as jnp
from jax import lax
from jax.experimental import pallas as pl
from jax.experimental.pallas import tpu as pltpu
from jax.experimental.pallas import tpu_sc as plsc

NUM_CORES = 2
NUM_SUBCORES = 16
NUM_WORKERS = NUM_CORES * NUM_SUBCORES
LANES = 16

B = 16384
F = 26
FPAD = 32
D = 32
NUM_FEATURES = 1000000
BPW = B // NUM_WORKERS
ROWS_PER_CHUNK = 16
CHUNKS = BPW // ROWS_PER_CHUNK
IDX_PER_CHUNK = ROWS_PER_CHUNK * F
RING = 4


def _fm_body(x_hbm, xp_hbm, w0_hbm, bias_hbm, emb_hbm, out_hbm,
             xv, xpv, w0v, bv, ebs, outv, esem, bsem):
  wid = lax.axis_index("s") * NUM_CORES + lax.axis_index("c")
  pltpu.sync_copy(
      x_hbm.at[pl.ds(pl.multiple_of(wid * BPW * F, 8), BPW * F)], xv)
  pltpu.sync_copy(
      xp_hbm.at[pl.ds(pl.multiple_of(wid * BPW * FPAD, 8), BPW * FPAD)], xpv)
  pltpu.sync_copy(w0_hbm, w0v)
  bias_cp = pltpu.make_async_copy(bias_hbm.at[xpv], bv, bsem)
  bias_cp.start()

  def emb_copy(c, slot):
    off = pl.multiple_of(c * IDX_PER_CHUNK, 8)
    return pltpu.make_async_copy(
        emb_hbm.at[xv.at[pl.ds(off, IDX_PER_CHUNK)]], ebs.at[slot],
        esem.at[slot])

  lane = lax.iota(jnp.int32, LANES)
  bias_mask = lane < (F - LANES)

  def compute(c, slot):
    res = jnp.zeros((LANES,), jnp.float32)
    for r in range(ROWS_PER_CHUNK):
      acc0 = jnp.zeros((LANES,), jnp.float32)
      acc1 = jnp.zeros((LANES,), jnp.float32)
      sq0 = jnp.zeros((LANES,), jnp.float32)
      sq1 = jnp.zeros((LANES,), jnp.float32)
      for f in range(F):
        row = r * F + f
        v0 = ebs[slot, row, pl.ds(0, LANES)]
        v1 = ebs[slot, row, pl.ds(LANES, LANES)]
        acc0 = acc0 + v0
        sq0 = sq0 + v0 * v0
        acc1 = acc1 + v1
        sq1 = sq1 + v1 * v1
      t = acc0 * acc0 + acc1 * acc1 - sq0 - sq1
      boff = pl.multiple_of((c * ROWS_PER_CHUNK + r) * FPAD, 8)
      b0 = bv[pl.ds(boff, LANES)]
      b1 = bv[pl.ds(boff + LANES, LANES)]
      u = 0.5 * t + b0 + jnp.where(bias_mask, b1, 0.0)
      total = jnp.sum(u)
      res = jnp.where(lane == r, total, res)
    outv[pl.ds(c * ROWS_PER_CHUNK, LANES)] = res + w0v[...]

  for c in range(RING):
    emb_copy(c, c).start()
  bias_cp.wait()

  @pl.loop(0, CHUNKS)
  def _chunk_loop(c):
    slot = lax.rem(c, RING)
    emb_copy(c, slot).wait()
    compute(c, slot)
    nxt = c + RING

    @pl.when(nxt < CHUNKS)
    def _():
      emb_copy(nxt, slot).start()

  pltpu.sync_copy(outv,
                  out_hbm.at[pl.ds(pl.multiple_of(wid * BPW, BPW), BPW)])


@jax.jit
def _fm_call(x_flat, xp_flat, w016, bias_flat, emb_table):
  return pl.kernel(
      _fm_body,
      out_type=jax.ShapeDtypeStruct((B,), jnp.float32),
      mesh=plsc.VectorSubcoreMesh(core_axis_name="c", subcore_axis_name="s"),
      compiler_params=pltpu.CompilerParams(
          needs_layout_passes=False, use_tc_tiling_on_sc=False),
      scratch_types=[
          pltpu.VMEM((BPW * F,), jnp.int32),
          pltpu.VMEM((BPW * FPAD,), jnp.int32),
          pltpu.VMEM((LANES,), jnp.float32),
          pltpu.VMEM((BPW * FPAD,), jnp.float32),
          pltpu.VMEM((RING, IDX_PER_CHUNK, D), jnp.float32),
          pltpu.VMEM((BPW,), jnp.float32),
          pltpu.SemaphoreType.DMA((RING,)),
          pltpu.SemaphoreType.DMA,
      ],
  )(x_flat, xp_flat, w016, bias_flat, emb_table)


def kernel(x, w0, bias_table, emb_table):
  x = x.astype(jnp.int32)
  xpad = jnp.pad(x, ((0, 0), (0, FPAD - F)))
  w016 = jnp.broadcast_to(w0.astype(jnp.float32), (LANES,))
  return _fm_call(x.reshape(-1), xpad.reshape(-1), w016,
                  bias_table.reshape(-1), emb_table)
